# baseline (speedup 1.0000x reference)
import functools

import jax
import jax.numpy as jnp
from jax import lax
from jax.experimental import pallas as pl
from jax.experimental.pallas import tpu as pltpu
from jax.experimental.pallas import tpu_sc as plsc

_R, _V = 128, 100000
_CHUNK = 10000
_NCHUNK = _V // _CHUNK
_NC, _NS = 2, 16
_NW = _NC * _NS
_ROWS_PER_W = _R // _NW

_CONSTS = {}


def _neg_log_expo():
    if "n" not in _CONSTS:
        e = jax.random.exponential(jax.random.key(42), (_R, _V), dtype=jnp.float32)
        n = jnp.minimum(-jnp.log(e), jnp.float32(3e37))
        _CONSTS["n"] = n.reshape(-1)
    return _CONSTS["n"]


def _body(logits_hbm, temps_hbm, noise_hbm, out_hbm, tbuf, lbuf, nbuf, resbuf, sem):
    wid = lax.axis_index("c") * _NS + lax.axis_index("s")
    pltpu.sync_copy(temps_hbm, tbuf.at[pl.ds(0, _R)])

    iota16 = lax.broadcasted_iota(jnp.int32, (16,), 0)
    res = jnp.zeros((16,), jnp.int32)

    for rlocal in range(_ROWS_PER_W):
        row = wid * _ROWS_PER_W + rlocal
        t_splat = jnp.full((16,), tbuf[pl.ds(row, 16)][0], jnp.float32)
        best = jnp.full((16,), -jnp.inf, jnp.float32)
        bidx = jnp.zeros((16,), jnp.int32)
        row_base = row * _V
        for chunk in range(_NCHUNK):
            off = row_base + chunk * _CHUNK
            pltpu.sync_copy(logits_hbm.at[pl.ds(off, _CHUNK)], lbuf)
            pltpu.sync_copy(noise_hbm.at[pl.ds(off, _CHUNK)], nbuf)

            def scan(i, carry, _chunk=chunk):
                b, bi = carry
                o = pl.multiple_of(i * 16, 16)
                l = lbuf[pl.ds(o, 16)]
                n = nbuf[pl.ds(o, 16)]
                s = l + t_splat * n
                idxv = iota16 + (_chunk * _CHUNK + i * 16)
                m = s > b
                return jnp.where(m, s, b), jnp.where(m, idxv, bi)

            best, bidx = lax.fori_loop(0, _CHUNK // 16, scan, (best, bidx))
        mx = jnp.max(best)
        cand = jnp.where(best == mx, bidx, jnp.int32(2**31 - 1))
        r = jnp.min(cand)
        res = jnp.where(iota16 == rlocal, r, res)

    resbuf[...] = res
    pltpu.sync_copy(resbuf, out_hbm.at[pl.ds(wid * 16, 16)])


@jax.jit
def _sampler(logits_flat, temps, noise_flat):
    mesh = plsc.VectorSubcoreMesh(core_axis_name="c", subcore_axis_name="s")
    k = functools.partial(
        pl.kernel,
        out_type=jax.ShapeDtypeStruct((_NW * 16,), jnp.int32),
        mesh=mesh,
        compiler_params=pltpu.CompilerParams(needs_layout_passes=False),
        scratch_types=[
            pltpu.VMEM((_R + 16,), jnp.float32),
            pltpu.VMEM((_CHUNK,), jnp.float32),
            pltpu.VMEM((_CHUNK,), jnp.float32),
            pltpu.VMEM((16,), jnp.int32),
            pltpu.SemaphoreType.DMA,
        ],
    )(_body)
    out = k(logits_flat, temps, noise_flat)
    return out.reshape(_NW, 16)[:, :_ROWS_PER_W].reshape(_R)


def kernel(logits, temperatures):
    noise = _neg_log_expo()
    return _sampler(logits.reshape(-1), temperatures, noise)


# speedup vs baseline: 1.1881x; 1.1881x over previous
import functools

import jax
import jax.numpy as jnp
import numpy as np
from jax import lax
from jax.experimental import pallas as pl
from jax.experimental.pallas import tpu as pltpu
from jax.experimental.pallas import tpu_sc as plsc

_R, _V = 128, 100000
_CHUNK = 20000
_NCHUNK = _V // _CHUNK
_CVREG = _CHUNK // 16
_ACC = 5
_NC, _NS = 2, 16
_NW = _NC * _NS
_ROWS_PER_W = _R // _NW
_BIG = np.int32(2**31 - 1)

_CONSTS = {}


def _neg_log_expo():
    if "n" not in _CONSTS:
        e = jax.random.exponential(jax.random.key(42), (_R, _V), dtype=jnp.float32)
        n = jnp.minimum(-jnp.log(e), jnp.float32(3e37))
        _CONSTS["n"] = n.reshape(-1)
    return _CONSTS["n"]


def _body(logits_hbm, temps_hbm, noise_hbm, out_hbm, tbuf, lbuf0, lbuf1,
          nbuf0, nbuf1, resbuf, lsem0, lsem1, nsem0, nsem1):
    wid = lax.axis_index("c") * _NS + lax.axis_index("s")
    pltpu.sync_copy(temps_hbm, tbuf.at[pl.ds(0, _R)])

    iota16 = lax.broadcasted_iota(jnp.int32, (16,), 0)
    res = jnp.zeros((16,), jnp.int32)
    row0 = wid * _ROWS_PER_W
    lbufs, nbufs = (lbuf0, lbuf1), (nbuf0, nbuf1)
    lsems, nsems = (lsem0, lsem1), (nsem0, nsem1)

    def fire(g):
        row, chunk = divmod(g, _NCHUNK)
        s = g & 1
        off = (row0 + row) * _V + chunk * _CHUNK
        hl = pltpu.async_copy(logits_hbm.at[pl.ds(off, _CHUNK)], lbufs[s], lsems[s])
        hn = pltpu.async_copy(noise_hbm.at[pl.ds(off, _CHUNK)], nbufs[s], nsems[s])
        return hl, hn

    ngl = _NCHUNK * _ROWS_PER_W
    pend = fire(0)
    for rlocal in range(_ROWS_PER_W):
        t_splat = jnp.full((16,), tbuf[pl.ds(row0 + rlocal, 16)][0], jnp.float32)
        best = [jnp.full((16,), -jnp.inf, jnp.float32) for _ in range(_ACC)]
        ci = [jnp.zeros((16,), jnp.int32) for _ in range(_ACC)]
        for chunk in range(_NCHUNK):
            g = rlocal * _NCHUNK + chunk
            s = g & 1
            nxt = fire(g + 1) if g + 1 < ngl else None
            pend[0].wait()
            pend[1].wait()
            pend = nxt
            lb, nb = lbufs[s], nbufs[s]
            base = chunk * _CVREG

            @plsc.parallel_loop(base, base + _CVREG, _ACC, unroll=2,
                                carry=tuple(best) + tuple(ci))
            def scan(i, carry, lb=lb, nb=nb, base=base, t_splat=t_splat):
                acc = list(carry)
                i_splat = jnp.full((16,), i, jnp.int32)
                for k in range(_ACC):
                    o = pl.multiple_of((i - base + k) * 16, 16)
                    s_val = lb[pl.ds(o, 16)] + t_splat * nb[pl.ds(o, 16)]
                    m = s_val > acc[k]
                    acc[k] = jnp.where(m, s_val, acc[k])
                    acc[_ACC + k] = jnp.where(m, i_splat, acc[_ACC + k])
                return tuple(acc)

            best, ci = list(scan[:_ACC]), list(scan[_ACC:])
        mx = best[0]
        for k in range(1, _ACC):
            mx = jnp.maximum(mx, best[k])
        mxs = jnp.max(mx)
        cand = _BIG
        for k in range(_ACC):
            idx = (ci[k] + k) * 16 + iota16
            cand = jnp.minimum(cand, jnp.where(best[k] == mxs, idx, _BIG))
        r = jnp.min(cand)
        res = jnp.where(iota16 == rlocal, r, res)

    resbuf[...] = res
    pltpu.sync_copy(resbuf, out_hbm.at[pl.ds(wid * 16, 16)])


@jax.jit
def _sampler(logits_flat, temps, noise_flat):
    mesh = plsc.VectorSubcoreMesh(core_axis_name="c", subcore_axis_name="s")
    k = functools.partial(
        pl.kernel,
        out_type=jax.ShapeDtypeStruct((_NW * 16,), jnp.int32),
        mesh=mesh,
        compiler_params=pltpu.CompilerParams(needs_layout_passes=False),
        scratch_types=[
            pltpu.VMEM((_R + 16,), jnp.float32),
            pltpu.VMEM((_CHUNK,), jnp.float32),
            pltpu.VMEM((_CHUNK,), jnp.float32),
            pltpu.VMEM((_CHUNK,), jnp.float32),
            pltpu.VMEM((_CHUNK,), jnp.float32),
            pltpu.VMEM((16,), jnp.int32),
            pltpu.SemaphoreType.DMA,
            pltpu.SemaphoreType.DMA,
            pltpu.SemaphoreType.DMA,
            pltpu.SemaphoreType.DMA,
        ],
    )(_body)
    out = k(logits_flat, temps, noise_flat)
    return out.reshape(_NW, 16)[:, :_ROWS_PER_W].reshape(_R)


def kernel(logits, temperatures):
    noise = _neg_log_expo()
    return _sampler(logits.reshape(-1), temperatures, noise)


# speedup vs baseline: 3.9959x; 3.3633x over previous
import functools

import jax
import jax.numpy as jnp
import numpy as np
from jax import lax
from jax.experimental import pallas as pl
from jax.experimental.pallas import tpu as pltpu
from jax.experimental.pallas import tpu_sc as plsc

_R, _V = 128, 100000
_CHUNK = 20000
_NCHUNK = _V // _CHUNK
_CVREG = _CHUNK // 16
_ACC = 5
_NC, _NS = 2, 16
_NW = _NC * _NS
_ROWS_PER_W = _R // _NW
_BIG = np.int32(2**31 - 1)

_CONSTS = {}


def _neg_log_expo():
    if "n" not in _CONSTS:
        with jax.ensure_compile_time_eval():
            e = jax.random.exponential(jax.random.key(42), (_R, _V), dtype=jnp.float32)
            n = jnp.minimum(-jnp.log(e), jnp.float32(3e37))
            _CONSTS["n"] = n.reshape(-1)
    return _CONSTS["n"]


def _body(logits_hbm, temps_hbm, noise_hbm, out_hbm, tbuf, lbuf0, lbuf1,
          nbuf0, nbuf1, resbuf, lsem0, lsem1, nsem0, nsem1):
    wid = lax.axis_index("c") * _NS + lax.axis_index("s")
    pltpu.sync_copy(temps_hbm, tbuf.at[pl.ds(0, _R)])

    iota16 = lax.broadcasted_iota(jnp.int32, (16,), 0)
    res = jnp.zeros((16,), jnp.int32)
    row0 = wid * _ROWS_PER_W
    lbufs, nbufs = (lbuf0, lbuf1), (nbuf0, nbuf1)
    lsems, nsems = (lsem0, lsem1), (nsem0, nsem1)

    def fire(g):
        row, chunk = divmod(g, _NCHUNK)
        s = g & 1
        off = (row0 + row) * _V + chunk * _CHUNK
        hl = pltpu.async_copy(logits_hbm.at[pl.ds(off, _CHUNK)], lbufs[s], lsems[s])
        hn = pltpu.async_copy(noise_hbm.at[pl.ds(off, _CHUNK)], nbufs[s], nsems[s])
        return hl, hn

    ngl = _NCHUNK * _ROWS_PER_W
    pend = fire(0)
    for rlocal in range(_ROWS_PER_W):
        t_splat = jnp.full((16,), tbuf[pl.ds(row0 + rlocal, 16)][0], jnp.float32)
        best = [jnp.full((16,), -jnp.inf, jnp.float32) for _ in range(_ACC)]
        ci = [jnp.zeros((16,), jnp.int32) for _ in range(_ACC)]
        for chunk in range(_NCHUNK):
            g = rlocal * _NCHUNK + chunk
            s = g & 1
            nxt = fire(g + 1) if g + 1 < ngl else None
            pend[0].wait()
            pend[1].wait()
            pend = nxt
            lb, nb = lbufs[s], nbufs[s]
            base = chunk * _CVREG

            @plsc.parallel_loop(base, base + _CVREG, _ACC, unroll=2,
                                carry=tuple(best) + tuple(ci))
            def scan(i, carry, lb=lb, nb=nb, base=base, t_splat=t_splat):
                acc = list(carry)
                i_splat = jnp.full((16,), i, jnp.int32)
                for k in range(_ACC):
                    o = pl.multiple_of((i - base + k) * 16, 16)
                    s_val = lb[pl.ds(o, 16)] + t_splat * nb[pl.ds(o, 16)]
                    m = s_val > acc[k]
                    acc[k] = jnp.where(m, s_val, acc[k])
                    acc[_ACC + k] = jnp.where(m, i_splat, acc[_ACC + k])
                return tuple(acc)

            best, ci = list(scan[:_ACC]), list(scan[_ACC:])
        mx = best[0]
        for k in range(1, _ACC):
            mx = jnp.maximum(mx, best[k])
        mxs = jnp.max(mx)
        cand = _BIG
        for k in range(_ACC):
            idx = (ci[k] + k) * 16 + iota16
            cand = jnp.minimum(cand, jnp.where(best[k] == mxs, idx, _BIG))
        r = jnp.min(cand)
        res = jnp.where(iota16 == rlocal, r, res)

    resbuf[...] = res
    pltpu.sync_copy(resbuf, out_hbm.at[pl.ds(wid * 16, 16)])


@jax.jit
def _sampler(logits_flat, temps, noise_flat):
    mesh = plsc.VectorSubcoreMesh(core_axis_name="c", subcore_axis_name="s")
    k = functools.partial(
        pl.kernel,
        out_type=jax.ShapeDtypeStruct((_NW * 16,), jnp.int32),
        mesh=mesh,
        compiler_params=pltpu.CompilerParams(needs_layout_passes=False),
        scratch_types=[
            pltpu.VMEM((_R + 16,), jnp.float32),
            pltpu.VMEM((_CHUNK,), jnp.float32),
            pltpu.VMEM((_CHUNK,), jnp.float32),
            pltpu.VMEM((_CHUNK,), jnp.float32),
            pltpu.VMEM((_CHUNK,), jnp.float32),
            pltpu.VMEM((16,), jnp.int32),
            pltpu.SemaphoreType.DMA,
            pltpu.SemaphoreType.DMA,
            pltpu.SemaphoreType.DMA,
            pltpu.SemaphoreType.DMA,
        ],
    )(_body)
    out = k(logits_flat, temps, noise_flat)
    return out.reshape(_NW, 16)[:, :_ROWS_PER_W].reshape(_R)


def kernel(logits, temperatures):
    noise = _neg_log_expo()
    return _sampler(logits.reshape(-1), temperatures, noise)


# speedup vs baseline: 5.3241x; 1.3324x over previous
import jax
import jax.numpy as jnp
import numpy as np
from jax import lax
from jax.experimental import pallas as pl
from jax.experimental.pallas import tpu as pltpu
from jax.experimental.pallas import tpu_sc as plsc

_R, _V = 128, 100000
_TILE = 128
_MAIN = (_V // _TILE) * _TILE
_TAILC = _V - _MAIN
_CT = 11 * _TILE
_NCHUNK = _MAIN // _CT
_CV = _CT // 16
_NBAND = _R // 8
_BIG = np.int32(2**31 - 1)
_NINF = np.float32(-np.inf)

_CONSTS = {}


def _noise_consts():
    def make():
        e = jax.random.exponential(jax.random.key(42), (_R, _V), dtype=jnp.float32)
        n = jnp.minimum(-jnp.log(e), jnp.float32(3e37))
        return n, n[:, _MAIN:].reshape(-1)

    if "n" not in _CONSTS:
        try:
            with jax.ensure_compile_time_eval():
                _CONSTS["n"] = make()
        except Exception:
            return make()
    return _CONSTS["n"]


def _merge_lanes(best, ci, iota16):
    mx = jnp.max(best)
    cand = jnp.where(best == mx, ci * 16 + iota16, _BIG)
    return mx, jnp.min(cand)


def _body(logits_hbm, temps_hbm, noise_hbm, tail_l_hbm, tail_n_hbm, out_hbm,
          tbuf, lbuf0, lbuf1, nbuf0, nbuf1, tlbuf, tnbuf,
          valbuf, idxbuf, pvalbuf, pidxbuf, resbuf, sh_v, sh_i,
          lsem0, lsem1, nsem0, nsem1, tsem):
    cid = lax.axis_index("c")
    sid = lax.axis_index("s")
    wid = cid * 16 + sid
    band = wid >> 1
    h = wid & 1
    row0 = pl.multiple_of(band * 8, 8)
    pltpu.sync_copy(temps_hbm, tbuf.at[pl.ds(0, _R)])

    iota16 = lax.broadcasted_iota(jnp.int32, (16,), 0)
    lbufs, nbufs = (lbuf0, lbuf1), (nbuf0, nbuf1)
    lsems, nsems = (lsem0, lsem1), (nsem0, nsem1)

    def fire(i):
        s = i & 1
        coff = pl.multiple_of((2 * i + h) * _CT, _TILE)
        src_l = logits_hbm.at[pl.ds(row0, 8), pl.ds(coff, _CT)]
        src_n = noise_hbm.at[pl.ds(row0, 8), pl.ds(coff, _CT)]
        return (pltpu.async_copy(src_l, lbufs[s], lsems[s]),
                pltpu.async_copy(src_n, nbufs[s], nsems[s]))

    t_splat = [jnp.full((16,), tbuf[pl.ds(row0 + s, 16)][0], jnp.float32)
               for s in range(8)]
    best = [jnp.full((16,), _NINF, jnp.float32) for _ in range(8)]
    ci = [jnp.zeros((16,), jnp.int32) for _ in range(8)]

    def scan_chunk(lb, nb, coff16, best, ci):
        @plsc.parallel_loop(0, _CV, 1, unroll=1, carry=tuple(best) + tuple(ci))
        def scan(v, carry):
            acc = list(carry)
            vg = jnp.full((16,), coff16 + v, jnp.int32)
            o = pl.multiple_of(v * 16, 16)
            for s in range(8):
                val = lb[s, pl.ds(o, 16)] + t_splat[s] * nb[s, pl.ds(o, 16)]
                m = val > acc[s]
                acc[s] = jnp.where(m, val, acc[s])
                acc[8 + s] = jnp.where(m, vg, acc[8 + s])
            return tuple(acc)
        return list(scan[:8]), list(scan[8:])

    pend = fire(0)
    for i in range(35):
        nxt = fire(i + 1) if i + 1 < 35 else None
        pend[0].wait()
        pend[1].wait()
        s = i & 1
        coff16 = (2 * i + h) * _CV
        best, ci = scan_chunk(lbufs[s], nbufs[s], coff16, best, ci)
        pend = nxt

    valbuf[...] = jnp.full((16,), _NINF, jnp.float32)
    idxbuf[...] = jnp.zeros((16,), jnp.int32)

    @pl.when(h == 0)
    def _():
        coff = pl.multiple_of(70 * _CT, _TILE)
        pltpu.sync_copy(logits_hbm.at[pl.ds(row0, 8), pl.ds(coff, _CT)], lbufs[0])
        pltpu.sync_copy(noise_hbm.at[pl.ds(row0, 8), pl.ds(coff, _CT)], nbufs[0])
        b70 = [jnp.full((16,), _NINF, jnp.float32) for _ in range(8)]
        c70 = [jnp.zeros((16,), jnp.int32) for _ in range(8)]
        b70, c70 = scan_chunk(lbufs[0], nbufs[0], 70 * _CV, b70, c70)
        v16 = jnp.full((16,), _NINF, jnp.float32)
        i16 = jnp.zeros((16,), jnp.int32)
        for s in range(8):
            mx, r = _merge_lanes(b70[s], c70[s], iota16)
            v16 = jnp.where(iota16 == s, mx, v16)
            i16 = jnp.where(iota16 == s, r, i16)
        valbuf[...] = v16
        idxbuf[...] = i16

    pltpu.async_copy(tail_l_hbm.at[pl.ds(band * 8 * _TAILC, 8 * _TAILC)],
                     tlbuf, tsem).wait()
    pltpu.async_copy(tail_n_hbm.at[pl.ds(band * 8 * _TAILC, 8 * _TAILC)],
                     tnbuf, tsem).wait()
    for v in range(_TAILC // 16):
        vg = jnp.full((16,), _MAIN // 16 + v, jnp.int32)
        for s in range(8):
            o = s * _TAILC + v * 16
            val = tlbuf[pl.ds(o, 16)] + t_splat[s] * tnbuf[pl.ds(o, 16)]
            m = val > best[s]
            best[s] = jnp.where(m, val, best[s])
            ci[s] = jnp.where(m, vg, ci[s])

    c70v = valbuf[...]
    c70i = idxbuf[...]
    val16 = jnp.full((16,), _NINF, jnp.float32)
    idx16 = jnp.zeros((16,), jnp.int32)
    for s in range(8):
        mx, r = _merge_lanes(best[s], ci[s], iota16)
        mvs = jnp.full((16,), mx, jnp.float32)
        rs = jnp.full((16,), r, jnp.int32)
        sel = iota16 == s
        val16 = jnp.where(sel, mvs, val16)
        idx16 = jnp.where(sel, rs, idx16)
    take70 = (c70v > val16) | ((c70v == val16) & (c70i < idx16))
    val16 = jnp.where(take70, c70v, val16)
    idx16 = jnp.where(take70, c70i, idx16)

    valbuf[...] = val16
    idxbuf[...] = idx16

    @pl.when(h == 1)
    def _():
        pltpu.sync_copy(valbuf, sh_v.at[pl.ds(sid * 16, 16)])
        pltpu.sync_copy(idxbuf, sh_i.at[pl.ds(sid * 16, 16)])

    plsc.subcore_barrier()

    @pl.when(h == 0)
    def _():
        pltpu.sync_copy(sh_v.at[pl.ds((sid + 1) * 16, 16)], pvalbuf)
        pltpu.sync_copy(sh_i.at[pl.ds((sid + 1) * 16, 16)], pidxbuf)
        bv = pvalbuf[...]
        bi = pidxbuf[...]
        takeb = (bv > val16) | ((bv == val16) & (bi < idx16))
        resbuf[...] = jnp.where(takeb, bi, idx16)
        pltpu.sync_copy(resbuf, out_hbm.at[pl.ds(band * 16, 16)])


@jax.jit
def _sampler(logits2d, temps, noise2d, tail_l, tail_n):
    mesh = plsc.VectorSubcoreMesh(core_axis_name="c", subcore_axis_name="s")
    k = pl.kernel(
        _body,
        out_type=jax.ShapeDtypeStruct((_NBAND * 16,), jnp.int32),
        mesh=mesh,
        compiler_params=pltpu.CompilerParams(needs_layout_passes=False),
        scratch_types=[
            pltpu.VMEM((_R + 16,), jnp.float32),
            pltpu.VMEM((8, _CT), jnp.float32),
            pltpu.VMEM((8, _CT), jnp.float32),
            pltpu.VMEM((8, _CT), jnp.float32),
            pltpu.VMEM((8, _CT), jnp.float32),
            pltpu.VMEM((8 * _TAILC,), jnp.float32),
            pltpu.VMEM((8 * _TAILC,), jnp.float32),
            pltpu.VMEM((16,), jnp.float32),
            pltpu.VMEM((16,), jnp.int32),
            pltpu.VMEM((16,), jnp.float32),
            pltpu.VMEM((16,), jnp.int32),
            pltpu.VMEM((16,), jnp.int32),
            pltpu.VMEM_SHARED((256,), jnp.float32),
            pltpu.VMEM_SHARED((256,), jnp.int32),
            pltpu.SemaphoreType.DMA,
            pltpu.SemaphoreType.DMA,
            pltpu.SemaphoreType.DMA,
            pltpu.SemaphoreType.DMA,
            pltpu.SemaphoreType.DMA,
        ],
    )
    out = k(logits2d, temps, noise2d, tail_l, tail_n)
    return out.reshape(_NBAND, 16)[:, :8].reshape(_R)


def kernel(logits, temperatures):
    noise2d, tail_n = _noise_consts()
    tail_l = logits[:, _MAIN:].reshape(-1)
    return _sampler(logits, temperatures, noise2d, tail_l, tail_n)


# speedup vs baseline: 7.9631x; 1.4957x over previous
import jax
import jax.numpy as jnp
import numpy as np
from jax import lax
from jax.experimental import pallas as pl
from jax.experimental.pallas import tpu as pltpu
from jax.experimental.pallas import tpu_sc as plsc

_R, _V = 128, 100000
_NT = _V // 8
_CTILE = 17
_NCHUNK = 23
_CVOC = _CTILE * 8
_NW = 32
_NINF = np.float32(-np.inf)

_CONSTS = {}


def _noise_t():
    def make():
        e = jax.random.exponential(jax.random.key(42), (_R, _V), dtype=jnp.float32)
        n = jnp.minimum(-jnp.log(e), jnp.float32(3e37))
        return n.T.copy()

    if "n" not in _CONSTS:
        try:
            with jax.ensure_compile_time_eval():
                _CONSTS["n"] = make()
        except Exception:
            return make()
    return _CONSTS["n"]


def _body(x_hbm, temps_hbm, noise_hbm, ov_hbm, oi_hbm,
          tbuf, lbuf0, lbuf1, nbuf0, nbuf1, vstage, istage, pvstage, pistage,
          sh_v, sh_i, lsem0, lsem1, nsem0, nsem1):
    cid = lax.axis_index("c")
    sid = lax.axis_index("s")
    wid = cid * 16 + sid
    pltpu.sync_copy(temps_hbm, tbuf)
    tvec = [tbuf[pl.ds(16 * k, 16)] for k in range(8)]

    start = 390 * wid + jnp.minimum(wid, 20)
    lbufs, nbufs = (lbuf0, lbuf1), (nbuf0, nbuf1)
    lsems, nsems = (lsem0, lsem1), (nsem0, nsem1)

    def fire(i):
        s = i & 1
        t0 = jnp.minimum(start + _CTILE * i, _NT - _CTILE)
        v0 = pl.multiple_of(t0 * 8, 8)
        return (pltpu.async_copy(x_hbm.at[pl.ds(v0, _CVOC)], lbufs[s], lsems[s]),
                pltpu.async_copy(noise_hbm.at[pl.ds(v0, _CVOC)], nbufs[s], nsems[s]),
                t0)

    best = [jnp.full((16,), _NINF, jnp.float32) for _ in range(8)]
    ci = [jnp.zeros((16,), jnp.int32) for _ in range(8)]

    pend = fire(0)
    for i in range(_NCHUNK):
        nxt = fire(i + 1) if i + 1 < _NCHUNK else None
        pend[0].wait()
        pend[1].wait()
        s = i & 1
        lb, nb = lbufs[s], nbufs[s]
        vbase = pend[2] * 8

        @plsc.parallel_loop(0, _CVOC, 1, unroll=1, carry=tuple(best) + tuple(ci))
        def scan(v, carry, lb=lb, nb=nb, vbase=vbase):
            acc = list(carry)
            vg = jnp.full((16,), vbase + v, jnp.int32)
            for k in range(8):
                val = lb[v, pl.ds(16 * k, 16)] + tvec[k] * nb[v, pl.ds(16 * k, 16)]
                m = val > acc[k]
                acc[k] = jnp.where(m, val, acc[k])
                acc[8 + k] = jnp.where(m, vg, acc[8 + k])
            return tuple(acc)

        best, ci = list(scan[:8]), list(scan[8:])
        pend = nxt

    for k in range(8):
        vstage[pl.ds(16 * k, 16)] = best[k]
        istage[pl.ds(16 * k, 16)] = ci[k]
    pltpu.sync_copy(vstage, sh_v.at[pl.ds(sid * 128, 128)])
    pltpu.sync_copy(istage, sh_i.at[pl.ds(sid * 128, 128)])
    plsc.subcore_barrier()

    @pl.when(sid == 0)
    def _():
        mv = [jnp.full((16,), _NINF, jnp.float32) for _ in range(8)]
        mi = [jnp.zeros((16,), jnp.int32) for _ in range(8)]
        for w in range(16):
            pltpu.sync_copy(sh_v.at[pl.ds(w * 128, 128)], pvstage)
            pltpu.sync_copy(sh_i.at[pl.ds(w * 128, 128)], pistage)
            for k in range(8):
                bv = pvstage[pl.ds(16 * k, 16)]
                bi = pistage[pl.ds(16 * k, 16)]
                take = (bv > mv[k]) | ((bv == mv[k]) & (bi < mi[k]))
                mv[k] = jnp.where(take, bv, mv[k])
                mi[k] = jnp.where(take, bi, mi[k])
        for k in range(8):
            vstage[pl.ds(16 * k, 16)] = mv[k]
            istage[pl.ds(16 * k, 16)] = mi[k]
        pltpu.sync_copy(vstage, ov_hbm.at[pl.ds(cid * 128, 128)])
        pltpu.sync_copy(istage, oi_hbm.at[pl.ds(cid * 128, 128)])


@jax.jit
def _sampler(x_t, temps, noise_t):
    mesh = plsc.VectorSubcoreMesh(core_axis_name="c", subcore_axis_name="s")
    k = pl.kernel(
        _body,
        out_type=(jax.ShapeDtypeStruct((2 * _R,), jnp.float32),
                  jax.ShapeDtypeStruct((2 * _R,), jnp.int32)),
        mesh=mesh,
        compiler_params=pltpu.CompilerParams(needs_layout_passes=False),
        scratch_types=[
            pltpu.VMEM((_R,), jnp.float32),
            pltpu.VMEM((_CVOC, _R), jnp.float32),
            pltpu.VMEM((_CVOC, _R), jnp.float32),
            pltpu.VMEM((_CVOC, _R), jnp.float32),
            pltpu.VMEM((_CVOC, _R), jnp.float32),
            pltpu.VMEM((_R,), jnp.float32),
            pltpu.VMEM((_R,), jnp.int32),
            pltpu.VMEM((_R,), jnp.float32),
            pltpu.VMEM((_R,), jnp.int32),
            pltpu.VMEM_SHARED((16 * _R,), jnp.float32),
            pltpu.VMEM_SHARED((16 * _R,), jnp.int32),
            pltpu.SemaphoreType.DMA,
            pltpu.SemaphoreType.DMA,
            pltpu.SemaphoreType.DMA,
            pltpu.SemaphoreType.DMA,
        ],
    )
    vals, idxs = k(x_t, temps, noise_t)
    v0, v1 = vals[:_R], vals[_R:]
    i0, i1 = idxs[:_R], idxs[_R:]
    take1 = (v1 > v0) | ((v1 == v0) & (i1 < i0))
    return jnp.where(take1, i1, i0)


def kernel(logits, temperatures):
    return _sampler(logits.T, temperatures, _noise_t())
